# baseline (speedup 1.0000x reference)
import jax
import jax.numpy as jnp
from jax.experimental import pallas as pl
from jax.experimental.pallas import tpu as pltpu

B = 16
L = 2048
Y = 2048
C = 32
H = 32
CH = C * H
TWO_PI = 6.283185307179586
T_CHUNK = 512
NCORE = 2
BPC = B // NCORE
NSLOT = 4


def _fanout_kernel(x_ref, w1et_ref, w2r_ref, b2_ref,
                   cx_ref, cy_ref, tx_ref, ty_ref,
                   ty_img, cy_img, per_s, sem_ty, sem_cy):
    pid = pl.program_id(0)

    ty_img[:, :L] = jnp.zeros((2 * C, L), jnp.float32)
    ty_img[:, L:] = jnp.ones((2 * C, Y), jnp.float32)
    for j in range(BPC):
        pltpu.make_async_copy(ty_img, ty_ref.at[pid * BPC + j], sem_ty.at[j]).start()

    i = jax.lax.broadcasted_iota(jnp.int32, (1, L + Y), 1)
    marks = jnp.where(i < L,
                      i.astype(jnp.float32) * (1.0 / L),
                      (i - L).astype(jnp.float32) * (1.0 / Y))
    marks2 = jnp.broadcast_to(marks, (BPC, L + Y))
    cx_ref[:, :] = marks2
    tx_ref[:, :] = marks2

    rowc = jax.lax.broadcasted_iota(jnp.int32, (C, CH), 0)
    coli = jax.lax.broadcasted_iota(jnp.int32, (C, CH), 1)
    mselt = jnp.where(coli // H == rowc, w2r_ref[:, :], 0.0)
    b2c = b2_ref[:, :]
    w1et = w1et_ref[:, :]
    for k in range(L // T_CHUNK):
        colt = jax.lax.broadcasted_iota(jnp.int32, (8, T_CHUNK), 1) + k * T_CHUNK
        rowi = jax.lax.broadcasted_iota(jnp.int32, (8, T_CHUNK), 0)
        phase = TWO_PI * (1.0 / L) * colt.astype(jnp.float32)
        phit = jnp.where(rowi == 0, jnp.sin(phase),
                         jnp.where(rowi == 1, jnp.cos(phase),
                                   jnp.where(rowi == 2, 1.0, 0.0)))
        ht = jnp.dot(w1et, phit, preferred_element_type=jnp.float32)
        ht = jnp.maximum(ht, 0.0)
        per = jnp.dot(mselt, ht, preferred_element_type=jnp.float32) + b2c
        per_s[:, pl.ds(k * T_CHUNK, T_CHUNK)] = per

    per = per_s[:, :]
    for j in range(BPC):
        s = j % NSLOT
        if j >= NSLOT:
            pltpu.make_async_copy(
                cy_img.at[s], cy_ref.at[pid * BPC + j - NSLOT],
                sem_cy.at[j - NSLOT]).wait()
        xt = jnp.transpose(x_ref[j, :, :], (1, 0))
        cy_img[s, :C, :L] = xt - per
        cy_img[s, C:, :L] = jnp.ones((C, L), jnp.float32)
        cy_img[s, :, L:] = jnp.zeros((2 * C, Y), jnp.float32)
        pltpu.make_async_copy(cy_img.at[s], cy_ref.at[pid * BPC + j],
                              sem_cy.at[j]).start()

    for j in range(BPC):
        pltpu.make_async_copy(ty_img, ty_ref.at[pid * BPC + j], sem_ty.at[j]).wait()
    for j in range(BPC - NSLOT, BPC):
        s = j % NSLOT
        pltpu.make_async_copy(cy_img.at[s], cy_ref.at[pid * BPC + j],
                              sem_cy.at[j]).wait()


@jax.jit
def kernel(x, W1, b1, W2, b2):
    w1f = W1.transpose(1, 0, 2).reshape(2, CH)
    b1f = b1.reshape(1, CH)
    w1e = jnp.concatenate([w1f, b1f, jnp.zeros((5, CH), jnp.float32)], axis=0)
    w1et = w1e.T
    w2r = W2.reshape(1, CH)

    out_shapes = (
        jax.ShapeDtypeStruct((B, L + Y), jnp.float32),
        jax.ShapeDtypeStruct((B, 2 * C, L + Y), jnp.float32),
        jax.ShapeDtypeStruct((B, L + Y), jnp.float32),
        jax.ShapeDtypeStruct((B, 2 * C, L + Y), jnp.float32),
    )
    in_specs = [
        pl.BlockSpec((BPC, L, C), lambda g: (g, 0, 0)),
        pl.BlockSpec((CH, 8), lambda g: (0, 0)),
        pl.BlockSpec((1, CH), lambda g: (0, 0)),
        pl.BlockSpec((C, 1), lambda g: (0, 0)),
    ]
    out_specs = (
        pl.BlockSpec((BPC, L + Y), lambda g: (g, 0)),
        pl.BlockSpec(memory_space=pl.ANY),
        pl.BlockSpec((BPC, L + Y), lambda g: (g, 0)),
        pl.BlockSpec(memory_space=pl.ANY),
    )
    cx, cy_t, tx, ty_t = pl.pallas_call(
        _fanout_kernel,
        grid=(NCORE,),
        in_specs=in_specs,
        out_specs=out_specs,
        out_shape=out_shapes,
        scratch_shapes=[
            pltpu.VMEM((2 * C, L + Y), jnp.float32),
            pltpu.VMEM((NSLOT, 2 * C, L + Y), jnp.float32),
            pltpu.VMEM((C, L), jnp.float32),
            pltpu.SemaphoreType.DMA((BPC,)),
            pltpu.SemaphoreType.DMA((BPC,)),
        ],
        compiler_params=pltpu.CompilerParams(
            dimension_semantics=("parallel",),
        ),
    )(x, w1et, w2r, b2)
    return (cx, jnp.transpose(cy_t, (0, 2, 1)), tx, jnp.transpose(ty_t, (0, 2, 1)))
